# baseline (speedup 1.0000x reference)
import functools

import jax
import jax.numpy as jnp
from jax import lax
from jax.experimental import pallas as pl
from jax.experimental.pallas import tpu as pltpu
from jax.experimental.pallas import tpu_sc as plsc

NUM_FIELDS = 26
VOCAB = 100000
DIM = 32
B = 4096


def _make_gather():
    mesh = plsc.VectorSubcoreMesh(core_axis_name="c", subcore_axis_name="s")

    @functools.partial(
        pl.kernel,
        mesh=mesh,
        out_type=jax.ShapeDtypeStruct((NUM_FIELDS, DIM, B), jnp.float32),
        compiler_params=pltpu.CompilerParams(needs_layout_passes=False),
        scratch_types=[
            pltpu.VMEM((8, 12544), jnp.float32),
            pltpu.VMEM((B,), jnp.int32),
            pltpu.VMEM((B,), jnp.float32),
            pltpu.SemaphoreType.DMA,
        ],
    )
    def gather_kernel(table_hbm, idx_hbm, out_hbm, slab_v, idx_v, out_v, sem):
        d = lax.axis_index("s") * 2 + lax.axis_index("c")
        tr = lax.rem(d, 4) * 8

        def per_field(f, _):
            pltpu.sync_copy(idx_hbm.at[f], idx_v)
            pltpu.async_copy(
                table_hbm.at[f, pl.ds(tr, 8), pl.ds(0, 12544)], slab_v, sem
            ).wait()
            pltpu.sync_copy(out_v, out_hbm.at[f, d])
            return _

        lax.fori_loop(0, NUM_FIELDS, per_field, None)

    return gather_kernel


_gather = _make_gather()


def kernel(all_inputs, tables):
    idx_t = all_inputs[:, 0, :].T
    tables_t = tables.transpose(0, 2, 1)
    out_t = _gather(tables_t, idx_t)
    return out_t.transpose(2, 0, 1)


# speedup vs baseline: 1.0739x; 1.0739x over previous
import functools

import jax
import jax.numpy as jnp
from jax import lax
from jax.experimental import pallas as pl
from jax.experimental.pallas import tpu as pltpu
from jax.experimental.pallas import tpu_sc as plsc

NUM_FIELDS = 26
VOCAB = 100000
DIM = 32
B = 4096

_LANES = 16
_H0 = 50048
_H1 = VOCAB - _H0


def _make_gather():
    mesh = plsc.VectorSubcoreMesh(core_axis_name="c", subcore_axis_name="s")

    @functools.partial(
        pl.kernel,
        mesh=mesh,
        out_type=jax.ShapeDtypeStruct((NUM_FIELDS, DIM, B), jnp.float32),
        compiler_params=pltpu.CompilerParams(needs_layout_passes=False),
        scratch_types=[
            pltpu.VMEM((_H0,), jnp.float32),
            pltpu.VMEM((_H1,), jnp.float32),
            pltpu.VMEM((B,), jnp.int32),
            pltpu.VMEM((B,), jnp.int32),
            pltpu.VMEM((B,), jnp.float32),
            pltpu.VMEM((B,), jnp.float32),
            pltpu.SemaphoreType.DMA,
            pltpu.SemaphoreType.DMA,
            pltpu.SemaphoreType.DMA,
            pltpu.SemaphoreType.DMA,
            pltpu.SemaphoreType.DMA,
        ],
    )
    def gather_kernel(table_hbm, idx_hbm, out_hbm,
                      h0_v, h1_v, idx_v0, idx_v1, out_v0, out_v1,
                      sem_h0, sem_h1, sem_idx, sem_o0, sem_o1):
        d = lax.axis_index("s") * 2 + lax.axis_index("c")
        idx_bufs = (idx_v0, idx_v1)
        out_bufs = (out_v0, out_v1)
        o_sems = (sem_o0, sem_o1)

        def half_copy(f, h):
            if h == 0:
                return pltpu.make_async_copy(
                    table_hbm.at[f, d, pl.ds(0, _H0)], h0_v, sem_h0)
            return pltpu.make_async_copy(
                table_hbm.at[f, d, pl.ds(_H0, _H1)], h1_v, sem_h1)

        def idx_copy(f, p):
            return pltpu.make_async_copy(idx_hbm.at[f, 0], idx_bufs[p], sem_idx)

        def out_copy(f, p):
            return pltpu.make_async_copy(out_bufs[p], out_hbm.at[f, d], o_sems[p])

        idx_copy(0, 0).start()
        half_copy(0, 0).start()
        half_copy(0, 1).start()

        def body(j, _):
            for ff in (0, 1):
                f = 2 * j + ff
                p, pn = ff, 1 - ff
                idx_v = idx_bufs[p]
                out_v = out_bufs[p]

                idx_copy(f, p).wait()

                @pl.when(f < NUM_FIELDS - 1)
                def _():
                    idx_copy(f + 1, pn).start()

                half_copy(f, 0).wait()

                @pl.when(f >= 2)
                def _():
                    out_copy(f - 2, p).wait()

                def pass0(c, _):
                    off = c * _LANES
                    vi = idx_v[pl.ds(off, _LANES)]
                    m = vi < _H0
                    out_v[pl.ds(off, _LANES)] = plsc.load_gather(
                        h0_v, [vi], mask=m)
                    return _

                lax.fori_loop(0, B // _LANES, pass0, None)

                @pl.when(f < NUM_FIELDS - 1)
                def _():
                    half_copy(f + 1, 0).start()

                half_copy(f, 1).wait()

                def pass1(c, _):
                    off = c * _LANES
                    vi = idx_v[pl.ds(off, _LANES)]
                    m = vi >= _H0
                    g = plsc.load_gather(h1_v, [vi - _H0], mask=m)
                    prev = out_v[pl.ds(off, _LANES)]
                    out_v[pl.ds(off, _LANES)] = jnp.where(m, g, prev)
                    return _

                lax.fori_loop(0, B // _LANES, pass1, None)

                @pl.when(f < NUM_FIELDS - 1)
                def _():
                    half_copy(f + 1, 1).start()

                out_copy(f, p).start()
            return _

        lax.fori_loop(0, NUM_FIELDS // 2, body, None)

        out_copy(NUM_FIELDS - 2, 0).wait()
        out_copy(NUM_FIELDS - 1, 1).wait()

    return gather_kernel


_gather = _make_gather()


def kernel(all_inputs, tables):
    idx_t = all_inputs.transpose(2, 1, 0)
    tables_t = tables.transpose(0, 2, 1)
    out_t = _gather(tables_t, idx_t)
    return out_t.transpose(2, 0, 1)
